# baseline (speedup 1.0000x reference)
import functools

import jax
import jax.numpy as jnp
from jax import lax
from jax.experimental import pallas as pl
from jax.experimental.pallas import tpu as pltpu
from jax.experimental.pallas import tpu_sc as plsc

_NUM_CORES = 2
_NUM_SUBCORES = 16
_CHUNK_ROWS = 16
_LANES = 16
_UNROLL = 16


def kernel(x, pe):
    batch, seq_len, d_model = x.shape
    n_rows = batch * seq_len
    n_workers = _NUM_CORES * _NUM_SUBCORES
    rows_per_w = n_rows // n_workers
    n_chunks = rows_per_w // _CHUNK_ROWS
    cw = _CHUNK_ROWS * d_model
    n_vregs = cw // _LANES

    xf = x.reshape(n_rows * d_model)
    pef = pe.reshape(seq_len * d_model)

    mesh = plsc.VectorSubcoreMesh(core_axis_name="c", subcore_axis_name="s")

    @functools.partial(
        pl.kernel,
        mesh=mesh,
        out_type=jax.ShapeDtypeStruct((n_rows * d_model,), x.dtype),
        scratch_types=[
            pltpu.VMEM((cw,), jnp.float32),
            pltpu.VMEM((cw,), jnp.float32),
            pltpu.VMEM((cw,), jnp.float32),
            pltpu.VMEM((cw,), jnp.float32),
            pltpu.VMEM((cw,), jnp.float32),
            pltpu.VMEM((cw,), jnp.float32),
            pltpu.SemaphoreType.DMA,
            pltpu.SemaphoreType.DMA,
            pltpu.SemaphoreType.DMA,
            pltpu.SemaphoreType.DMA,
            pltpu.SemaphoreType.DMA,
            pltpu.SemaphoreType.DMA,
        ],
    )
    def sc_add(x_hbm, pe_hbm, out_hbm, xb0, xb1, pb0, pb1, ob0, ob1,
               xs0, xs1, ps0, ps1, os0, os1):
        wid = lax.axis_index("s") * _NUM_CORES + lax.axis_index("c")
        base = wid * (rows_per_w * d_model)
        rows_per_batch = seq_len
        b_idx = (wid * rows_per_w) // rows_per_batch
        pe_base = (wid * rows_per_w - b_idx * rows_per_batch) * d_model

        xb = (xb0, xb1)
        pb = (pb0, pb1)
        ob = (ob0, ob1)
        xs = (xs0, xs1)
        ps = (ps0, ps1)
        osem = (os0, os1)

        def x_src(j):
            return x_hbm.at[pl.ds(base + j * cw, cw)]

        def pe_src(j):
            return pe_hbm.at[pl.ds(pe_base + j * cw, cw)]

        def out_dst(j):
            return out_hbm.at[pl.ds(base + j * cw, cw)]

        for b in range(2):
            pltpu.async_copy(x_src(b), xb[b], xs[b])
            pltpu.async_copy(pe_src(b), pb[b], ps[b])

        def outer(jj, carry):
            for b in range(2):
                j = jj * 2 + b
                pltpu.make_async_copy(x_src(j), xb[b], xs[b]).wait()
                pltpu.make_async_copy(pe_src(j), pb[b], ps[b]).wait()
                @pl.when(j >= 2)
                def _():
                    pltpu.make_async_copy(ob[b], out_dst(j - 2), osem[b]).wait()

                def add_body(i, c):
                    off = i * (_UNROLL * _LANES)
                    for k in range(_UNROLL):
                        sl = pl.ds(off + k * _LANES, _LANES)
                        ob[b][sl] = xb[b][sl] + pb[b][sl]
                    return c

                lax.fori_loop(0, n_vregs // _UNROLL, add_body, 0)

                @pl.when(j + 2 < n_chunks)
                def _():
                    pltpu.async_copy(x_src(j + 2), xb[b], xs[b])
                    pltpu.async_copy(pe_src(j + 2), pb[b], ps[b])

                pltpu.async_copy(ob[b], out_dst(j), osem[b])
            return carry

        lax.fori_loop(0, n_chunks // 2, outer, 0)

        for b in range(2):
            j = n_chunks - 2 + b
            pltpu.make_async_copy(ob[b], out_dst(j), osem[b]).wait()

    out = sc_add(xf, pef)
    return out.reshape(batch, seq_len, d_model)


# speedup vs baseline: 2.5466x; 2.5466x over previous
import functools

import jax
import jax.numpy as jnp
from jax import lax
from jax.experimental import pallas as pl
from jax.experimental.pallas import tpu as pltpu
from jax.experimental.pallas import tpu_sc as plsc

_NUM_CORES = 2
_NUM_SUBCORES = 16
_CHUNK_ROWS = 16
_LANES = 16


def kernel(x, pe):
    batch, seq_len, d_model = x.shape
    n_rows = batch * seq_len
    n_workers = _NUM_CORES * _NUM_SUBCORES
    rows_per_w = n_rows // n_workers
    n_chunks = rows_per_w // _CHUNK_ROWS
    vregs_per_row = d_model // _LANES

    xf = x.reshape(n_rows, d_model)

    mesh = plsc.VectorSubcoreMesh(core_axis_name="c", subcore_axis_name="s")

    @functools.partial(
        pl.kernel,
        mesh=mesh,
        out_type=jax.ShapeDtypeStruct((n_rows, d_model), x.dtype),
        scratch_types=[
            pltpu.VMEM((_CHUNK_ROWS, d_model), jnp.float32),
            pltpu.VMEM((_CHUNK_ROWS, d_model), jnp.float32),
            pltpu.VMEM((_CHUNK_ROWS, d_model), jnp.float32),
            pltpu.VMEM((_CHUNK_ROWS, d_model), jnp.float32),
            pltpu.VMEM((_CHUNK_ROWS, d_model), jnp.float32),
            pltpu.VMEM((_CHUNK_ROWS, d_model), jnp.float32),
            pltpu.SemaphoreType.DMA,
            pltpu.SemaphoreType.DMA,
            pltpu.SemaphoreType.DMA,
            pltpu.SemaphoreType.DMA,
            pltpu.SemaphoreType.DMA,
            pltpu.SemaphoreType.DMA,
        ],
    )
    def sc_add(x_hbm, pe_hbm, out_hbm, xb0, xb1, pb0, pb1, ob0, ob1,
               xs0, xs1, ps0, ps1, os0, os1):
        wid = lax.axis_index("s") * _NUM_CORES + lax.axis_index("c")
        base = wid * rows_per_w
        b_idx = base // seq_len
        pe_base = base - b_idx * seq_len

        xb = (xb0, xb1)
        pb = (pb0, pb1)
        ob = (ob0, ob1)
        xs = (xs0, xs1)
        ps = (ps0, ps1)
        osem = (os0, os1)

        def x_src(j):
            return x_hbm.at[pl.ds(base + j * _CHUNK_ROWS, _CHUNK_ROWS)]

        def pe_src(j):
            return pe_hbm.at[pl.ds(pe_base + j * _CHUNK_ROWS, _CHUNK_ROWS)]

        def out_dst(j):
            return out_hbm.at[pl.ds(base + j * _CHUNK_ROWS, _CHUNK_ROWS)]

        for b in range(2):
            pltpu.async_copy(x_src(b), xb[b], xs[b])
            pltpu.async_copy(pe_src(b), pb[b], ps[b])

        def outer(jj, carry):
            for b in range(2):
                j = jj * 2 + b
                pltpu.make_async_copy(x_src(j), xb[b], xs[b]).wait()
                pltpu.make_async_copy(pe_src(j), pb[b], ps[b]).wait()
                @pl.when(j >= 2)
                def _():
                    pltpu.make_async_copy(ob[b], out_dst(j - 2), osem[b]).wait()

                def add_body(r, c):
                    for k in range(vregs_per_row):
                        sl = pl.ds(k * _LANES, _LANES)
                        ob[b][r, sl] = xb[b][r, sl] + pb[b][r, sl]
                    return c

                lax.fori_loop(0, _CHUNK_ROWS, add_body, 0)

                @pl.when(j + 2 < n_chunks)
                def _():
                    pltpu.async_copy(x_src(j + 2), xb[b], xs[b])
                    pltpu.async_copy(pe_src(j + 2), pb[b], ps[b])

                pltpu.async_copy(ob[b], out_dst(j), osem[b])
            return carry

        lax.fori_loop(0, n_chunks // 2, outer, 0)

        for b in range(2):
            j = n_chunks - 2 + b
            pltpu.make_async_copy(ob[b], out_dst(j), osem[b]).wait()

    out = sc_add(xf, pe)
    return out.reshape(batch, seq_len, d_model)


# speedup vs baseline: 2.8616x; 1.1237x over previous
import functools

import jax
import jax.numpy as jnp
from jax import lax
from jax.experimental import pallas as pl
from jax.experimental.pallas import tpu as pltpu
from jax.experimental.pallas import tpu_sc as plsc

_NUM_CORES = 2
_NUM_SUBCORES = 16
_CHUNK_ROWS = 16
_LANES = 16


def kernel(x, pe):
    batch, seq_len, d_model = x.shape
    n_rows = batch * seq_len
    n_workers = _NUM_CORES * _NUM_SUBCORES
    s_per_w = seq_len // n_workers
    n_chunks = s_per_w // _CHUNK_ROWS
    vregs_per_row = d_model // _LANES

    xf = x.reshape(n_rows, d_model)

    mesh = plsc.VectorSubcoreMesh(core_axis_name="c", subcore_axis_name="s")

    @functools.partial(
        pl.kernel,
        mesh=mesh,
        out_type=jax.ShapeDtypeStruct((n_rows, d_model), x.dtype),
        scratch_types=[
            pltpu.VMEM((_CHUNK_ROWS, d_model), jnp.float32),
            pltpu.VMEM((_CHUNK_ROWS, d_model), jnp.float32),
            pltpu.VMEM((_CHUNK_ROWS, d_model), jnp.float32),
            pltpu.VMEM((_CHUNK_ROWS, d_model), jnp.float32),
            pltpu.VMEM((_CHUNK_ROWS, d_model), jnp.float32),
            pltpu.VMEM((_CHUNK_ROWS, d_model), jnp.float32),
            pltpu.SemaphoreType.DMA,
            pltpu.SemaphoreType.DMA,
            pltpu.SemaphoreType.DMA,
            pltpu.SemaphoreType.DMA,
            pltpu.SemaphoreType.DMA,
            pltpu.SemaphoreType.DMA,
        ],
    )
    def sc_add(x_hbm, pe_hbm, out_hbm, xb0, xb1, pb0, pb1, ob0, ob1,
               xs0, xs1, ps0, ps1, os0, os1):
        wid = lax.axis_index("s") * _NUM_CORES + lax.axis_index("c")
        s_base = wid * s_per_w

        xb = (xb0, xb1)
        pb = (pb0, pb1)
        ob = (ob0, ob1)
        xs = (xs0, xs1)
        ps = (ps0, ps1)
        osem = (os0, os1)

        n_items = n_chunks * batch

        def x_src(j, b):
            return x_hbm.at[pl.ds(b * seq_len + s_base + j * _CHUNK_ROWS,
                                  _CHUNK_ROWS)]

        def pe_src(j):
            return pe_hbm.at[pl.ds(s_base + j * _CHUNK_ROWS, _CHUNK_ROWS)]

        def out_dst(j, b):
            return out_hbm.at[pl.ds(b * seq_len + s_base + j * _CHUNK_ROWS,
                                    _CHUNK_ROWS)]

        for p in range(2):
            pltpu.async_copy(pe_src(p), pb[p], ps[p])
            pltpu.async_copy(x_src(0, p), xb[p], xs[p])

        def item(j, b, slot, pslot, first, last):
            if first:
                pltpu.make_async_copy(pe_src(j), pb[pslot], ps[pslot]).wait()
            i = j * batch + b
            pltpu.make_async_copy(x_src(j, b), xb[slot], xs[slot]).wait()

            @pl.when(i >= 2)
            def _():
                jp, bp = (i - 2) // batch, (i - 2) % batch
                pltpu.make_async_copy(ob[slot], out_dst(jp, bp),
                                      osem[slot]).wait()

            def add_body(r, c):
                for k in range(vregs_per_row):
                    sl = pl.ds(k * _LANES, _LANES)
                    ob[slot][r, sl] = xb[slot][r, sl] + pb[pslot][r, sl]
                return c

            lax.fori_loop(0, _CHUNK_ROWS, add_body, 0)

            @pl.when(i + 2 < n_items)
            def _():
                jn, bn = (i + 2) // batch, (i + 2) % batch
                pltpu.async_copy(x_src(jn, bn), xb[slot], xs[slot])

            if last:
                @pl.when(j + 2 < n_chunks)
                def _():
                    pltpu.async_copy(pe_src(j + 2), pb[pslot], ps[pslot])

            pltpu.async_copy(ob[slot], out_dst(j, b), osem[slot])

        def outer(j2, carry):
            for jo in range(2):
                j = j2 * 2 + jo
                for b in range(batch):
                    item(j, b, slot=b % 2, pslot=jo,
                         first=(b == 0), last=(b == batch - 1))
            return carry

        lax.fori_loop(0, n_chunks // 2, outer, 0)

        for b in range(batch - 2, batch):
            pltpu.make_async_copy(ob[b % 2], out_dst(n_chunks - 1, b),
                                  osem[b % 2]).wait()

    out = sc_add(xf, pe)
    return out.reshape(batch, seq_len, d_model)


# speedup vs baseline: 2.9438x; 1.0287x over previous
import functools

import jax
import jax.numpy as jnp
from jax import lax
from jax.experimental import pallas as pl
from jax.experimental.pallas import tpu as pltpu
from jax.experimental.pallas import tpu_sc as plsc

_NUM_CORES = 2
_NUM_SUBCORES = 16
_CHUNK_ROWS = 8
_LANES = 16


def kernel(x, pe):
    batch, seq_len, d_model = x.shape
    n_rows = batch * seq_len
    n_workers = _NUM_CORES * _NUM_SUBCORES
    s_per_w = seq_len // n_workers
    n_chunks = s_per_w // _CHUNK_ROWS
    vregs_per_row = d_model // _LANES

    xf = x.reshape(n_rows, d_model)

    mesh = plsc.VectorSubcoreMesh(core_axis_name="c", subcore_axis_name="s")

    row_chunk = pltpu.VMEM((_CHUNK_ROWS, d_model), jnp.float32)

    @functools.partial(
        pl.kernel,
        mesh=mesh,
        out_type=jax.ShapeDtypeStruct((n_rows, d_model), x.dtype),
        scratch_types=(
            [row_chunk] * batch
            + [row_chunk] * 2
            + [row_chunk] * batch
            + [pltpu.SemaphoreType.DMA] * batch
            + [pltpu.SemaphoreType.DMA] * 2
            + [pltpu.SemaphoreType.DMA] * batch
        ),
    )
    def sc_add(x_hbm, pe_hbm, out_hbm, *refs):
        xb = refs[0:batch]
        pb = refs[batch:batch + 2]
        ob = refs[batch + 2:2 * batch + 2]
        xs = refs[2 * batch + 2:3 * batch + 2]
        ps = refs[3 * batch + 2:3 * batch + 4]
        osem = refs[3 * batch + 4:4 * batch + 4]

        wid = lax.axis_index("s") * _NUM_CORES + lax.axis_index("c")
        s_base = wid * s_per_w

        def x_src(j, b):
            return x_hbm.at[pl.ds(b * seq_len + s_base + j * _CHUNK_ROWS,
                                  _CHUNK_ROWS)]

        def pe_src(j):
            return pe_hbm.at[pl.ds(s_base + j * _CHUNK_ROWS, _CHUNK_ROWS)]

        def out_dst(j, b):
            return out_hbm.at[pl.ds(b * seq_len + s_base + j * _CHUNK_ROWS,
                                    _CHUNK_ROWS)]

        for p in range(2):
            pltpu.async_copy(pe_src(p), pb[p], ps[p])
        for b in range(batch):
            pltpu.async_copy(x_src(0, b), xb[b], xs[b])

        def chunk_body(j, pslot):
            pltpu.make_async_copy(pe_src(j), pb[pslot], ps[pslot]).wait()
            for b in range(batch):
                pltpu.make_async_copy(x_src(j, b), xb[b], xs[b]).wait()

                @pl.when(j >= 1)
                def _():
                    pltpu.make_async_copy(ob[b], out_dst(j - 1, b),
                                          osem[b]).wait()

                def add_body(r, c):
                    for k in range(vregs_per_row):
                        sl = pl.ds(k * _LANES, _LANES)
                        ob[b][r, sl] = xb[b][r, sl] + pb[pslot][r, sl]
                    return c

                lax.fori_loop(0, _CHUNK_ROWS, add_body, 0)

                @pl.when(j + 1 < n_chunks)
                def _():
                    pltpu.async_copy(x_src(j + 1, b), xb[b], xs[b])

                if b == batch - 1:
                    @pl.when(j + 2 < n_chunks)
                    def _():
                        pltpu.async_copy(pe_src(j + 2), pb[pslot], ps[pslot])

                pltpu.async_copy(ob[b], out_dst(j, b), osem[b])

        def outer(j2, carry):
            for jo in range(2):
                chunk_body(j2 * 2 + jo, jo)
            return carry

        lax.fori_loop(0, n_chunks // 2, outer, 0)

        for b in range(batch):
            pltpu.make_async_copy(ob[b], out_dst(n_chunks - 1, b),
                                  osem[b]).wait()

    out = sc_add(xf, pe)
    return out.reshape(batch, seq_len, d_model)
